# baseline (speedup 1.0000x reference)
import jax
import jax.numpy as jnp
from jax.experimental import pallas as pl
from jax.experimental.pallas import tpu as pltpu

_CHUNK = 512
_NBUF = 3


def _copy(adj_hbm, bufs, sems, chunk_idx, slot):
    return pltpu.make_async_copy(
        adj_hbm.at[pl.ds(chunk_idx * _CHUNK, _CHUNK), :],
        bufs.at[slot],
        sems.at[slot],
    )


def _fused_gnn_kernel(feat_ref, w_ref, adj_hbm, out_ref, support_ref, bufs, sems):
    i = pl.program_id(0)
    nch = pl.num_programs(0)
    slot = jax.lax.rem(i, _NBUF)

    @pl.when(i == 0)
    def _():
        for b in range(_NBUF):
            _copy(adj_hbm, bufs, sems, jnp.int32(b), jnp.int32(b)).start()
        support_ref[...] = jnp.dot(
            feat_ref[...], w_ref[...], preferred_element_type=jnp.float32
        )

    _copy(adj_hbm, bufs, sems, i, slot).wait()
    out_ref[...] = jnp.maximum(
        jnp.dot(bufs[slot], support_ref[...], preferred_element_type=jnp.float32),
        0.0,
    )

    @pl.when(i + _NBUF < nch)
    def _():
        _copy(adj_hbm, bufs, sems, i + _NBUF, slot).start()


def kernel(features, adj, weight):
    n, d_in = features.shape
    d_out = weight.shape[1]
    return pl.pallas_call(
        _fused_gnn_kernel,
        grid=(n // _CHUNK,),
        in_specs=[
            pl.BlockSpec((n, d_in), lambda i: (0, 0)),
            pl.BlockSpec((d_in, d_out), lambda i: (0, 0)),
            pl.BlockSpec(memory_space=pl.ANY),
        ],
        out_specs=pl.BlockSpec((_CHUNK, d_out), lambda i: (i, 0)),
        out_shape=jax.ShapeDtypeStruct((n, d_out), jnp.float32),
        scratch_shapes=[
            pltpu.VMEM((n, d_out), jnp.float32),
            pltpu.VMEM((_NBUF, _CHUNK, n), jnp.float32),
            pltpu.SemaphoreType.DMA((_NBUF,)),
        ],
    )(features, weight, adj)


# speedup vs baseline: 1.0419x; 1.0419x over previous
import jax
import jax.numpy as jnp
from jax.experimental import pallas as pl
from jax.experimental.pallas import tpu as pltpu

_BLOCK = 512


def _fused_gnn_kernel(feat_ref, w_ref, adj_ref, out_ref, support_ref):
    @pl.when(pl.program_id(0) == 0)
    def _():
        support_ref[...] = jnp.dot(
            feat_ref[...], w_ref[...], preferred_element_type=jnp.float32
        )

    out_ref[...] = jnp.maximum(
        jnp.dot(adj_ref[...], support_ref[...], preferred_element_type=jnp.float32),
        0.0,
    )


def kernel(features, adj, weight):
    n, d_in = features.shape
    d_out = weight.shape[1]
    return pl.pallas_call(
        _fused_gnn_kernel,
        grid=(n // _BLOCK,),
        in_specs=[
            pl.BlockSpec((n, d_in), lambda i: (0, 0)),
            pl.BlockSpec((d_in, d_out), lambda i: (0, 0)),
            pl.BlockSpec((_BLOCK, n), lambda i: (i, 0)),
        ],
        out_specs=pl.BlockSpec((_BLOCK, d_out), lambda i: (i, 0)),
        out_shape=jax.ShapeDtypeStruct((n, d_out), jnp.float32),
        scratch_shapes=[pltpu.VMEM((n, d_out), jnp.float32)],
    )(features, weight, adj)
